# baseline (speedup 1.0000x reference)
import functools

import jax
import jax.numpy as jnp
from jax import lax
from jax.experimental import pallas as pl
from jax.experimental.pallas import tpu as pltpu
from jax.experimental.pallas import tpu_sc as plsc

_E = 8
_K = 2
_B = 256
_NBP = 64
_NC, _NS, _L = 2, 16, 16
_NW = _NC * _NS


def _gate_kernel(x_ref, wg_ref, bg_ref, e0_ref, e1_ref, w0_ref, w1_ref):
    xb = x_ref[...]
    logits = (jnp.dot(xb, wg_ref[...], preferred_element_type=jnp.float32)
              + bg_ref[...])
    tb = xb.shape[0]
    eids = lax.broadcasted_iota(jnp.int32, (tb, _E), 1)
    m0 = jnp.max(logits, axis=1, keepdims=True)
    i0 = jnp.min(jnp.where(logits == m0, eids, _E), axis=1, keepdims=True)
    l2 = jnp.where(eids == i0, -jnp.inf, logits)
    m1 = jnp.max(l2, axis=1, keepdims=True)
    i1 = jnp.min(jnp.where(l2 == m1, eids, _E), axis=1, keepdims=True)
    t = jnp.exp(m1 - m0)
    e0_ref[...] = i0
    e1_ref[...] = i1
    w0_ref[...] = 1.0 / (1.0 + t)
    w1_ref[...] = t / (1.0 + t)


def _gating(x2, Wg, bg):
    n, d = x2.shape
    tb = 512
    return pl.pallas_call(
        _gate_kernel,
        grid=(n // tb,),
        in_specs=[
            pl.BlockSpec((tb, d), lambda i: (i, 0)),
            pl.BlockSpec((d, _E), lambda i: (0, 0)),
            pl.BlockSpec((1, _E), lambda i: (0, 0)),
        ],
        out_specs=[pl.BlockSpec((tb, 1), lambda i: (i, 0))] * 4,
        out_shape=[
            jax.ShapeDtypeStruct((n, 1), jnp.int32),
            jax.ShapeDtypeStruct((n, 1), jnp.int32),
            jax.ShapeDtypeStruct((n, 1), jnp.float32),
            jax.ShapeDtypeStruct((n, 1), jnp.float32),
        ],
    )(x2, Wg, bg.reshape(1, _E))


def _route_kernel(pairs_ref, slot_ref, bexp_ref):
    p = pairs_ref.shape[0]
    e = pairs_ref[...]
    onehot = (e == lax.broadcasted_iota(jnp.int32, (p, _E), 1)).astype(
        jnp.int32)
    csum = onehot
    sh = 1
    while sh < p:
        csum = csum + jnp.concatenate(
            [jnp.zeros((sh, _E), jnp.int32), csum[:-sh, :]], axis=0)
        sh *= 2
    rank = jnp.sum(onehot * csum, axis=1, keepdims=True) - 1
    counts = csum[p - 1:p, :]
    pc = ((counts + (_B - 1)) // _B) * _B
    lt = (lax.broadcasted_iota(jnp.int32, (_E, _E), 0)
          < lax.broadcasted_iota(jnp.int32, (_E, _E), 1)).astype(jnp.float32)
    start = jnp.dot(pc.astype(jnp.float32), lt,
                    preferred_element_type=jnp.float32).astype(jnp.int32)
    slot_ref[...] = jnp.sum(onehot * start, axis=1, keepdims=True) + rank
    endi = start + pc
    jb = lax.broadcasted_iota(jnp.int32, (_NBP, 1), 0) * _B
    be = jnp.sum((endi <= jb).astype(jnp.int32), axis=1, keepdims=True)
    bexp_ref[...] = jnp.minimum(be, _E - 1)


def _routing(pairs):
    p = pairs.shape[0]
    return pl.pallas_call(
        _route_kernel,
        out_shape=[
            jax.ShapeDtypeStruct((p, 1), jnp.int32),
            jax.ShapeDtypeStruct((_NBP, 1), jnp.int32),
        ],
    )(pairs)


def _dispatch(slot_f, wflat, x2, s_total):
    p = slot_f.shape[0]
    n_tok, d = x2.shape
    sw = s_total // _NW
    tw = n_tok // _NW
    ch = 32
    nch = tw // ch
    mesh = plsc.VectorSubcoreMesh(core_axis_name="c", subcore_axis_name="s")

    @functools.partial(
        pl.kernel,
        out_type=[
            jax.ShapeDtypeStruct((s_total, d), jnp.float32),
            jax.ShapeDtypeStruct((s_total,), jnp.float32),
        ],
        mesh=mesh,
        compiler_params=pltpu.CompilerParams(needs_layout_passes=False),
        scratch_types=[
            pltpu.VMEM((p,), jnp.int32),
            pltpu.VMEM((p,), jnp.float32),
            pltpu.VMEM((sw,), jnp.float32),
            pltpu.VMEM((nch, ch), jnp.int32),
            pltpu.VMEM((nch, ch), jnp.int32),
            pltpu.VMEM((ch, d), jnp.float32),
            pltpu.VMEM((ch, d), jnp.float32),
            pltpu.SemaphoreType.DMA,
            pltpu.SemaphoreType.DMA,
            pltpu.SemaphoreType.DMA,
            pltpu.SemaphoreType.DMA,
        ],
    )
    def dk(slot_hbm, w_hbm, x_hbm, xg_hbm, wv_hbm,
           slots_v, w_v, lw, sidx0, sidx1, xb0, xb1, r0, r1, s0, s1):
        wid = lax.axis_index("s") * _NC + lax.axis_index("c")
        lo = wid * sw
        t0 = wid * tw
        pltpu.sync_copy(slot_hbm, slots_v)
        pltpu.sync_copy(w_hbm, w_v)

        def zbody(i, carry):
            lw[pl.ds(i * _L, _L)] = jnp.zeros((_L,), jnp.float32)
            return carry
        lax.fori_loop(0, sw // _L, zbody, 0)

        def sbody(g, carry):
            sv = slots_v[pl.ds(g * _L, _L)]
            m = (sv >= lo) & (sv < lo + sw)
            idx = jnp.where(m, sv - lo, 0)
            plsc.store_scatter(lw, [idx], w_v[pl.ds(g * _L, _L)], mask=m)
            return carry
        lax.fori_loop(0, p // _L, sbody, 0)

        pltpu.sync_copy(lw, wv_hbm.at[pl.ds(lo, sw)])

        for c in range(nch):
            pltpu.sync_copy(slot_hbm.at[pl.ds(t0 + c * ch, ch)], sidx0.at[c])
            pltpu.sync_copy(slot_hbm.at[pl.ds(n_tok + t0 + c * ch, ch)],
                            sidx1.at[c])

        bufs = [xb0, xb1]
        rs = [r0, r1]
        ss = [s0, s1]
        robj = [None, None]
        sobj = [None, None]
        robj[0] = pltpu.async_copy(x_hbm.at[pl.ds(t0, ch)], bufs[0], rs[0])
        for c in range(nch):
            b = c & 1
            b2 = 1 - b
            if c + 1 < nch:
                if sobj[b2] is not None:
                    sobj[b2][0].wait()
                    sobj[b2][1].wait()
                robj[b2] = pltpu.async_copy(
                    x_hbm.at[pl.ds(t0 + (c + 1) * ch, ch)], bufs[b2], rs[b2])
            robj[b].wait()
            sobj[b] = (
                pltpu.async_copy(bufs[b], xg_hbm.at[sidx0.at[c]], ss[b]),
                pltpu.async_copy(bufs[b], xg_hbm.at[sidx1.at[c]], ss[b]),
            )
        for b in range(2):
            if sobj[b] is not None:
                sobj[b][0].wait()
                sobj[b][1].wait()

    return dk(slot_f, wflat, x2)


def _ffn_kernel(bexp_ref, xg_ref, w1_ref, b1_ref, w2_ref, b2_ref, wv_ref,
                y_ref):
    xb = xg_ref[...].astype(jnp.bfloat16)
    h = (jnp.dot(xb, w1_ref[0], preferred_element_type=jnp.float32)
         + b1_ref[0])
    h = jax.nn.gelu(h).astype(jnp.bfloat16)
    y = (jnp.dot(h, w2_ref[0], preferred_element_type=jnp.float32)
         + b2_ref[0])
    y_ref[...] = y * jnp.reshape(wv_ref[0], (_B, 1))


def _ffn(bexp, xg, W1, b1, W2, b2, wv, nb):
    s_total, d = xg.shape
    f = W1.shape[2]
    grid_spec = pltpu.PrefetchScalarGridSpec(
        num_scalar_prefetch=1,
        grid=(nb,),
        in_specs=[
            pl.BlockSpec((_B, d), lambda j, be: (j, 0)),
            pl.BlockSpec((1, d, f), lambda j, be: (be[j], 0, 0)),
            pl.BlockSpec((1, 1, f), lambda j, be: (be[j], 0, 0)),
            pl.BlockSpec((1, f, d), lambda j, be: (be[j], 0, 0)),
            pl.BlockSpec((1, 1, d), lambda j, be: (be[j], 0, 0)),
            pl.BlockSpec((1, 1, _B), lambda j, be: (j, 0, 0)),
        ],
        out_specs=pl.BlockSpec((_B, d), lambda j, be: (j, 0)),
    )
    return pl.pallas_call(
        _ffn_kernel,
        grid_spec=grid_spec,
        out_shape=jax.ShapeDtypeStruct((s_total, d), jnp.float32),
    )(bexp, xg, W1, b1.reshape(_E, 1, f), W2, b2.reshape(_E, 1, d),
      wv.reshape(nb, 1, _B))


def _combine(y, sl0, sl1):
    n_tok = sl0.shape[0]
    d = y.shape[1]
    tw = n_tok // _NW
    ch = _L
    nch = tw // ch
    mesh = plsc.VectorSubcoreMesh(core_axis_name="c", subcore_axis_name="s")

    @functools.partial(
        pl.kernel,
        out_type=jax.ShapeDtypeStruct((n_tok, d), jnp.float32),
        mesh=mesh,
        scratch_types=[
            pltpu.VMEM((nch, ch), jnp.int32),
            pltpu.VMEM((nch, ch), jnp.int32),
            pltpu.VMEM((ch, d), jnp.float32),
            pltpu.VMEM((ch, d), jnp.float32),
            pltpu.VMEM((ch, d), jnp.float32),
            pltpu.VMEM((ch, d), jnp.float32),
            pltpu.VMEM((ch, d), jnp.float32),
            pltpu.VMEM((ch, d), jnp.float32),
            pltpu.SemaphoreType.DMA,
            pltpu.SemaphoreType.DMA,
            pltpu.SemaphoreType.DMA,
            pltpu.SemaphoreType.DMA,
            pltpu.SemaphoreType.DMA,
            pltpu.SemaphoreType.DMA,
        ],
    )
    def ck(y_hbm, sl0_hbm, sl1_hbm, out_hbm,
           sl0v, sl1v, b0a, b1a, oba, b0b, b1b, obb,
           g0a, g1a, g0b, g1b, wsa, wsb):
        wid = lax.axis_index("s") * _NC + lax.axis_index("c")
        t0 = wid * tw
        for c in range(nch):
            pltpu.sync_copy(sl0_hbm.at[pl.ds(t0 + c * ch, ch)], sl0v.at[c])
            pltpu.sync_copy(sl1_hbm.at[pl.ds(t0 + c * ch, ch)], sl1v.at[c])
        b0 = [b0a, b0b]
        b1 = [b1a, b1b]
        ob = [oba, obb]
        g0 = [g0a, g0b]
        g1 = [g1a, g1b]
        wsm = [wsa, wsb]
        gobj = [None, None]
        wobj = [None, None]

        def issue(c, s):
            gobj[s] = (
                pltpu.async_copy(y_hbm.at[sl0v.at[c]], b0[s], g0[s]),
                pltpu.async_copy(y_hbm.at[sl1v.at[c]], b1[s], g1[s]),
            )

        issue(0, 0)
        for c in range(nch):
            s = c & 1
            s2 = 1 - s
            if c + 1 < nch:
                if wobj[s2] is not None:
                    wobj[s2].wait()
                issue(c + 1, s2)
            ga, gb = gobj[s]
            ga.wait()
            gb.wait()

            def cbody(i, carry, s=s):
                fl = i * 4
                for u in range(4):
                    r = (fl + u) >> 6
                    cc = (fl + u) & 63
                    sl = pl.ds(cc * _L, _L)
                    ob[s][r, sl] = b0[s][r, sl] + b1[s][r, sl]
                return carry
            lax.fori_loop(0, ch * (d // _L) // 4, cbody, 0)
            wobj[s] = pltpu.async_copy(
                ob[s], out_hbm.at[pl.ds(t0 + c * ch, ch)], wsm[s])
        wobj[0].wait()
        wobj[1].wait()

    return ck(y, sl0, sl1)


def kernel(x, Wg, bg, W1, b1, W2, b2):
    n = x.shape[0] * x.shape[1]
    d = x.shape[2]
    x2 = x.reshape(n, d)

    e0, e1, w0, w1 = _gating(x2, Wg, bg)

    pairs = jnp.concatenate([e0, e1], axis=0)
    slot, bexp = _routing(pairs)
    slot_f = slot.reshape(2 * n)
    wflat = jnp.concatenate([w0, w1], axis=0).reshape(2 * n)

    nb = (n * _K) // _B + _E
    s_total = nb * _B
    xg, wv = _dispatch(slot_f, wflat, x2, s_total)

    y = _ffn(bexp.reshape(_NBP), xg, W1.astype(jnp.bfloat16), b1,
             W2.astype(jnp.bfloat16), b2, wv, nb)

    out = _combine(y, slot_f[:n], slot_f[n:])
    return out.reshape(x.shape)


# speedup vs baseline: 1.1942x; 1.1942x over previous
import functools

import jax
import jax.numpy as jnp
from jax import lax
from jax.experimental import pallas as pl
from jax.experimental.pallas import tpu as pltpu
from jax.experimental.pallas import tpu_sc as plsc

_E = 8
_K = 2
_B = 256
_NBP = 64
_NC, _NS, _L = 2, 16, 16
_NW = _NC * _NS


def _gateroute(x2, Wg, bg):
    n, d = x2.shape
    tb = 512
    nblk = n // tb
    p = n * _K

    def body(x_ref, wg_ref, bg_ref, slot_ref, bexp_ref, wflat_ref,
             pr_scr, w_scr):
        i = pl.program_id(0)

        @pl.when(i < nblk)
        def _gate():
            xb = x_ref[...]
            logits = (jnp.dot(xb, wg_ref[...],
                              preferred_element_type=jnp.float32)
                      + bg_ref[...])
            eids = lax.broadcasted_iota(jnp.int32, (tb, _E), 1)
            m0 = jnp.max(logits, axis=1, keepdims=True)
            i0 = jnp.min(jnp.where(logits == m0, eids, _E), axis=1,
                         keepdims=True)
            l2 = jnp.where(eids == i0, -jnp.inf, logits)
            m1 = jnp.max(l2, axis=1, keepdims=True)
            i1 = jnp.min(jnp.where(l2 == m1, eids, _E), axis=1,
                         keepdims=True)
            t = jnp.exp(m1 - m0)
            pr_scr[pl.ds(i * tb, tb), :] = i0
            pr_scr[pl.ds(n + i * tb, tb), :] = i1
            w_scr[pl.ds(i * tb, tb), :] = 1.0 / (1.0 + t)
            w_scr[pl.ds(n + i * tb, tb), :] = t / (1.0 + t)

        @pl.when(i == nblk)
        def _route():
            e = pr_scr[...]
            onehot = (e == lax.broadcasted_iota(jnp.int32, (p, _E), 1)
                      ).astype(jnp.int32)
            csum = onehot
            sh = 1
            while sh < p:
                csum = csum + jnp.concatenate(
                    [jnp.zeros((sh, _E), jnp.int32), csum[:-sh, :]], axis=0)
                sh *= 2
            rank = jnp.sum(onehot * csum, axis=1, keepdims=True) - 1
            counts = csum[p - 1:p, :]
            pc = ((counts + (_B - 1)) // _B) * _B
            lt = (lax.broadcasted_iota(jnp.int32, (_E, _E), 0)
                  < lax.broadcasted_iota(jnp.int32, (_E, _E), 1)
                  ).astype(jnp.float32)
            st = jnp.dot(pc.astype(jnp.float32), lt,
                         preferred_element_type=jnp.float32).astype(jnp.int32)
            slot_ref[...] = jnp.sum(onehot * st, axis=1, keepdims=True) + rank
            endi = st + pc
            jb = lax.broadcasted_iota(jnp.int32, (_NBP, 1), 0) * _B
            be = jnp.sum((endi <= jb).astype(jnp.int32), axis=1,
                         keepdims=True)
            bexp_ref[...] = jnp.minimum(be, _E - 1)
            wflat_ref[...] = w_scr[...]

    return pl.pallas_call(
        body,
        grid=(nblk + 1,),
        in_specs=[
            pl.BlockSpec((tb, d), lambda i: (jnp.minimum(i, nblk - 1), 0)),
            pl.BlockSpec((d, _E), lambda i: (0, 0)),
            pl.BlockSpec((1, _E), lambda i: (0, 0)),
        ],
        out_specs=[
            pl.BlockSpec((p, 1), lambda i: (0, 0)),
            pl.BlockSpec((_NBP, 1), lambda i: (0, 0)),
            pl.BlockSpec((p, 1), lambda i: (0, 0)),
        ],
        out_shape=[
            jax.ShapeDtypeStruct((p, 1), jnp.int32),
            jax.ShapeDtypeStruct((_NBP, 1), jnp.int32),
            jax.ShapeDtypeStruct((p, 1), jnp.float32),
        ],
        scratch_shapes=[
            pltpu.VMEM((p, 1), jnp.int32),
            pltpu.VMEM((p, 1), jnp.float32),
        ],
    )(x2, Wg, bg.reshape(1, _E))


def _dispatch(slot_f, wflat, x2, s_total):
    p = slot_f.shape[0]
    n_tok, d = x2.shape
    sw = s_total // _NW
    tw = n_tok // _NW
    ch = 32
    nch = tw // ch
    mesh = plsc.VectorSubcoreMesh(core_axis_name="c", subcore_axis_name="s")

    @functools.partial(
        pl.kernel,
        out_type=[
            jax.ShapeDtypeStruct((s_total, d), jnp.float32),
            jax.ShapeDtypeStruct((s_total,), jnp.float32),
        ],
        mesh=mesh,
        compiler_params=pltpu.CompilerParams(needs_layout_passes=False),
        scratch_types=[
            pltpu.VMEM((p,), jnp.int32),
            pltpu.VMEM((p,), jnp.float32),
            pltpu.VMEM((sw,), jnp.float32),
            pltpu.VMEM((nch, ch), jnp.int32),
            pltpu.VMEM((nch, ch), jnp.int32),
            pltpu.VMEM((ch, d), jnp.float32),
            pltpu.VMEM((ch, d), jnp.float32),
            pltpu.SemaphoreType.DMA,
            pltpu.SemaphoreType.DMA,
            pltpu.SemaphoreType.DMA,
            pltpu.SemaphoreType.DMA,
        ],
    )
    def dk(slot_hbm, w_hbm, x_hbm, xg_hbm, wv_hbm,
           slots_v, w_v, lw, sidx0, sidx1, xb0, xb1, r0, r1, s0, s1):
        wid = lax.axis_index("s") * _NC + lax.axis_index("c")
        lo = wid * sw
        t0 = wid * tw
        for c in range(nch):
            pltpu.sync_copy(slot_hbm.at[pl.ds(t0 + c * ch, ch)], sidx0.at[c])
            pltpu.sync_copy(slot_hbm.at[pl.ds(n_tok + t0 + c * ch, ch)],
                            sidx1.at[c])
        pltpu.sync_copy(slot_hbm, slots_v)
        pltpu.sync_copy(w_hbm, w_v)

        def zbody(i, carry):
            lw[pl.ds(i * _L, _L)] = jnp.zeros((_L,), jnp.float32)
            return carry
        lax.fori_loop(0, sw // _L, zbody, 0)

        def sbody(g, carry):
            sv = slots_v[pl.ds(g * _L, _L)]
            m = (sv >= lo) & (sv < lo + sw)
            idx = jnp.where(m, sv - lo, 0)
            plsc.store_scatter(lw, [idx], w_v[pl.ds(g * _L, _L)], mask=m)
            return carry

        bufs = [xb0, xb1]
        rs = [r0, r1]
        ss = [s0, s1]
        robj = [None, None]
        sobj = [None, None]
        scan_per = (p // _L) // nch
        robj[0] = pltpu.async_copy(x_hbm.at[pl.ds(t0, ch)], bufs[0], rs[0])
        for c in range(nch):
            b = c & 1
            b2 = 1 - b
            if c + 1 < nch:
                if sobj[b2] is not None:
                    sobj[b2][0].wait()
                    sobj[b2][1].wait()
                robj[b2] = pltpu.async_copy(
                    x_hbm.at[pl.ds(t0 + (c + 1) * ch, ch)], bufs[b2], rs[b2])
            robj[b].wait()
            sobj[b] = (
                pltpu.async_copy(bufs[b], xg_hbm.at[sidx0.at[c]], ss[b]),
                pltpu.async_copy(bufs[b], xg_hbm.at[sidx1.at[c]], ss[b]),
            )
            lax.fori_loop(c * scan_per, (c + 1) * scan_per, sbody, 0)
        pltpu.sync_copy(lw, wv_hbm.at[pl.ds(lo, sw)])
        for b in range(2):
            if sobj[b] is not None:
                sobj[b][0].wait()
                sobj[b][1].wait()

    return dk(slot_f, wflat, x2)


def _ffn_kernel(bexp_ref, xg_ref, w1_ref, b1_ref, w2_ref, b2_ref, wv_ref,
                y_ref):
    h = (jnp.dot(xg_ref[...], w1_ref[0], preferred_element_type=jnp.float32)
         + b1_ref[0])
    h = jax.nn.gelu(h)
    y = (jnp.dot(h, w2_ref[0], preferred_element_type=jnp.float32)
         + b2_ref[0])
    y_ref[...] = y * jnp.reshape(wv_ref[0], (_B, 1))


def _ffn(bexp, xg, W1, b1, W2, b2, wv, nb):
    s_total, d = xg.shape
    f = W1.shape[2]
    grid_spec = pltpu.PrefetchScalarGridSpec(
        num_scalar_prefetch=1,
        grid=(nb,),
        in_specs=[
            pl.BlockSpec((_B, d), lambda j, be: (j, 0)),
            pl.BlockSpec((1, d, f), lambda j, be: (be[j], 0, 0)),
            pl.BlockSpec((1, 1, f), lambda j, be: (be[j], 0, 0)),
            pl.BlockSpec((1, f, d), lambda j, be: (be[j], 0, 0)),
            pl.BlockSpec((1, 1, d), lambda j, be: (be[j], 0, 0)),
            pl.BlockSpec((1, 1, _B), lambda j, be: (j, 0, 0)),
        ],
        out_specs=pl.BlockSpec((_B, d), lambda j, be: (j, 0)),
    )
    return pl.pallas_call(
        _ffn_kernel,
        grid_spec=grid_spec,
        out_shape=jax.ShapeDtypeStruct((s_total, d), jnp.float32),
    )(bexp, xg, W1, b1.reshape(_E, 1, f), W2, b2.reshape(_E, 1, d),
      wv.reshape(nb, 1, _B))


def _combine(y, sl0, sl1):
    n_tok = sl0.shape[0]
    d = y.shape[1]
    tw = n_tok // _NW
    ch = _L
    nch = tw // ch
    mesh = plsc.VectorSubcoreMesh(core_axis_name="c", subcore_axis_name="s")

    @functools.partial(
        pl.kernel,
        out_type=jax.ShapeDtypeStruct((n_tok, d), jnp.float32),
        mesh=mesh,
        scratch_types=[
            pltpu.VMEM((nch, ch), jnp.int32),
            pltpu.VMEM((nch, ch), jnp.int32),
            pltpu.VMEM((ch, d), jnp.float32),
            pltpu.VMEM((ch, d), jnp.float32),
            pltpu.VMEM((ch, d), jnp.float32),
            pltpu.VMEM((ch, d), jnp.float32),
            pltpu.VMEM((ch, d), jnp.float32),
            pltpu.VMEM((ch, d), jnp.float32),
            pltpu.SemaphoreType.DMA,
            pltpu.SemaphoreType.DMA,
            pltpu.SemaphoreType.DMA,
            pltpu.SemaphoreType.DMA,
            pltpu.SemaphoreType.DMA,
            pltpu.SemaphoreType.DMA,
        ],
    )
    def ck(y_hbm, sl0_hbm, sl1_hbm, out_hbm,
           sl0v, sl1v, b0a, b1a, oba, b0b, b1b, obb,
           g0a, g1a, g0b, g1b, wsa, wsb):
        wid = lax.axis_index("s") * _NC + lax.axis_index("c")
        t0 = wid * tw
        for c in range(nch):
            pltpu.sync_copy(sl0_hbm.at[pl.ds(t0 + c * ch, ch)], sl0v.at[c])
            pltpu.sync_copy(sl1_hbm.at[pl.ds(t0 + c * ch, ch)], sl1v.at[c])
        b0 = [b0a, b0b]
        b1 = [b1a, b1b]
        ob = [oba, obb]
        g0 = [g0a, g0b]
        g1 = [g1a, g1b]
        wsm = [wsa, wsb]
        gobj = [None, None]
        wobj = [None, None]

        def issue(c, s):
            gobj[s] = (
                pltpu.async_copy(y_hbm.at[sl0v.at[c]], b0[s], g0[s]),
                pltpu.async_copy(y_hbm.at[sl1v.at[c]], b1[s], g1[s]),
            )

        issue(0, 0)
        for c in range(nch):
            s = c & 1
            s2 = 1 - s
            if c + 1 < nch:
                if wobj[s2] is not None:
                    wobj[s2].wait()
                issue(c + 1, s2)
            ga, gb = gobj[s]
            ga.wait()
            gb.wait()

            def cbody(i, carry, s=s):
                fl = i * 8
                for u in range(8):
                    r = (fl + u) >> 6
                    cc = (fl + u) & 63
                    sl = pl.ds(cc * _L, _L)
                    ob[s][r, sl] = b0[s][r, sl] + b1[s][r, sl]
                return carry
            lax.fori_loop(0, ch * (d // _L) // 8, cbody, 0)
            wobj[s] = pltpu.async_copy(
                ob[s], out_hbm.at[pl.ds(t0 + c * ch, ch)], wsm[s])
        wobj[0].wait()
        wobj[1].wait()

    return ck(y, sl0, sl1)


def kernel(x, Wg, bg, W1, b1, W2, b2):
    n = x.shape[0] * x.shape[1]
    d = x.shape[2]
    x2 = x.reshape(n, d)

    slot, bexp, wflat2 = _gateroute(x2, Wg, bg)
    slot_f = slot.reshape(2 * n)
    wflat = wflat2.reshape(2 * n)

    nb = (n * _K) // _B + _E
    s_total = nb * _B
    xg, wv = _dispatch(slot_f, wflat, x2, s_total)

    y = _ffn(bexp.reshape(_NBP), xg, W1, b1, W2, b2, wv, nb)

    out = _combine(y, slot_f[:n], slot_f[n:])
    return out.reshape(x.shape)
